# baseline (speedup 1.0000x reference)
import functools

import jax
import jax.numpy as jnp
from jax import lax
from jax.experimental import pallas as pl
from jax.experimental.pallas import tpu as pltpu
from jax.experimental.pallas import tpu_sc as plsc

NP = 10240
CH = 128
NCORES = 2
NSUB = 16
NW = NCORES * NSUB
ROWS_PER_TILE = NP // NSUB
ZR = 16
QB = 2


def _sc_mesh():
    return plsc.VectorSubcoreMesh(core_axis_name="c", subcore_axis_name="s")


def _zero_acc(acc, zbuf, sid, c):
    zvec = jnp.zeros((16,), jnp.float32)

    def fill(r, _):
        for t in range(c // 16):
            zbuf[r, pl.ds(t * 16, 16)] = zvec
        return 0

    lax.fori_loop(0, ZR, fill, 0)
    base_row = sid * ROWS_PER_TILE
    for t in range(ROWS_PER_TILE // ZR):
        pltpu.sync_copy(zbuf, acc.at[pl.ds(base_row + t * ZR, ZR)])


def _make_deg_kernel(EP):
    NCH = EP // (NW * CH)

    @functools.partial(
        pl.kernel,
        out_type=jax.ShapeDtypeStruct((NCORES * NP, 128), jnp.float32),
        mesh=_sc_mesh(),
        scratch_types=[
            pltpu.VMEM_SHARED((NP, 128), jnp.float32),
            pltpu.VMEM((NCH * CH,), jnp.int32),
            pltpu.VMEM((CH, 128), jnp.float32),
            pltpu.VMEM((ZR, 128), jnp.float32),
        ]
        + [pltpu.SemaphoreType.DMA] * QB,
    )
    def k(dst_hbm, val_hbm, out_hbm, acc, dsti, valb, zbuf, *ss):
        cid = lax.axis_index("c")
        sid = lax.axis_index("s")
        _zero_acc(acc, zbuf, sid, 128)
        pltpu.sync_copy(val_hbm, valb)
        wid = sid * NCORES + cid
        pltpu.sync_copy(dst_hbm.at[pl.ds(wid * NCH * CH, NCH * CH)], dsti)
        plsc.subcore_barrier()

        def pipe(p, _):
            base = p * QB
            sds = [
                pltpu.async_copy(
                    valb, acc.at[dsti.at[pl.ds((base + b) * CH, CH)]], ss[b], add=True
                )
                for b in range(QB)
            ]
            for d in sds:
                d.wait()
            return 0

        lax.fori_loop(0, NCH // QB, pipe, 0)
        plsc.subcore_barrier()
        base_row = sid * ROWS_PER_TILE
        pltpu.sync_copy(
            acc.at[pl.ds(base_row, ROWS_PER_TILE)],
            out_hbm.at[pl.ds(cid * NP + base_row, ROWS_PER_TILE)],
        )

    return k


def _make_scatter_kernel(C, EP):
    NCH = EP // (NW * CH)

    @functools.partial(
        pl.kernel,
        out_type=jax.ShapeDtypeStruct((NCORES * NP, C), jnp.float32),
        mesh=_sc_mesh(),
        scratch_types=[
            pltpu.VMEM_SHARED((NP, C), jnp.float32),
            pltpu.VMEM((CH,), jnp.int32),
            pltpu.VMEM((CH,), jnp.int32),
            pltpu.VMEM((CH, C), jnp.float32),
            pltpu.VMEM((ZR, C), jnp.float32),
            pltpu.SemaphoreType.DMA,
        ],
    )
    def k(g_hbm, src_hbm, dst_hbm, out_hbm, acc, srcb, dstb, rows, zbuf, sem):
        cid = lax.axis_index("c")
        sid = lax.axis_index("s")
        _zero_acc(acc, zbuf, sid, C)
        wid = sid * NCORES + cid
        ebase = wid * NCH * CH
        plsc.subcore_barrier()

        def body(ci, _):
            off = ebase + ci * CH
            pltpu.sync_copy(src_hbm.at[pl.ds(off, CH)], srcb)
            pltpu.sync_copy(dst_hbm.at[pl.ds(off, CH)], dstb)
            pltpu.async_copy(g_hbm.at[srcb], rows, sem).wait()
            pltpu.sync_copy(rows, acc.at[dstb], add=True)
            return 0

        lax.fori_loop(0, NCH, body, 0)
        plsc.subcore_barrier()
        base_row = sid * ROWS_PER_TILE
        pltpu.sync_copy(
            acc.at[pl.ds(base_row, ROWS_PER_TILE)],
            out_hbm.at[pl.ds(cid * NP + base_row, ROWS_PER_TILE)],
        )

    return k


_R = 256


def _dinv(c0, c1):
    return lax.rsqrt(c0[:, 0:1] + c1[:, 0:1] + 1.0)


def _tc_scale(c0_ref, c1_ref, x_ref, o_ref):
    o_ref[...] = x_ref[...] * _dinv(c0_ref[...], c1_ref[...])


def _tc_mid(c0_ref, c1_ref, a0_ref, a1_ref, g_ref, o_ref):
    d = _dinv(c0_ref[...], c1_ref[...])
    o_ref[...] = (d * d) * (a0_ref[...] + a1_ref[...] + g_ref[...])


def _tc_mlp(c0_ref, c1_ref, a0_ref, a1_ref, g_ref, w1_ref, b1_ref, w2_ref, o_ref):
    d = _dinv(c0_ref[...], c1_ref[...])
    p = d * (a0_ref[...] + a1_ref[...] + g_ref[...])
    hid = jnp.maximum(
        jnp.dot(p, w1_ref[...], preferred_element_type=jnp.float32) + b1_ref[...], 0.0
    )
    o_ref[...] = d * jnp.dot(hid, w2_ref[...], preferred_element_type=jnp.float32)


def _tc_final(c0_ref, c1_ref, a0_ref, a1_ref, g_ref, b2_ref, o_ref):
    d = _dinv(c0_ref[...], c1_ref[...])
    out_c = b2_ref.shape[1]
    s = (d * (a0_ref[...] + a1_ref[...] + g_ref[...]))[:, :out_c] + b2_ref[...]
    m = jnp.max(s, axis=1, keepdims=True)
    e = jnp.exp(s - m)
    lse = jnp.log(jnp.sum(e, axis=1, keepdims=True))
    o_ref[...] = s - m - lse


def _row_spec(c):
    return pl.BlockSpec((_R, c), lambda i: (i, 0))


def _full_spec(r, c):
    return pl.BlockSpec((r, c), lambda i: (0, 0))


def _call_rows(body, in_specs, out_c, args):
    grid = NP // _R
    return pl.pallas_call(
        body,
        grid=(grid,),
        in_specs=in_specs,
        out_specs=_row_spec(out_c),
        out_shape=jax.ShapeDtypeStruct((NP, out_c), jnp.float32),
    )(*args)


def kernel(x, edge_index, W1, b1, W2, b2):
    N, in_c = x.shape
    E = edge_index.shape[1]
    hid_c = W1.shape[1]
    out_c = W2.shape[1]

    EG = NW * CH * QB
    EP = ((E + EG - 1) // EG) * EG

    xp = jnp.zeros((NP, in_c), jnp.float32).at[:N].set(x)
    pad = jnp.full((EP - E,), N, jnp.int32)
    srcp = jnp.concatenate([edge_index[0], pad])
    dstp = jnp.concatenate([edge_index[1], pad])
    W2p = jnp.zeros((hid_c, hid_c), jnp.float32).at[:, :out_c].set(W2)

    deg_k = _make_deg_kernel(EP)
    scat_big = _make_scatter_kernel(in_c, EP)

    ones_val = jnp.zeros((CH, 128), jnp.float32).at[:, 0].set(1.0)
    cnt = deg_k(dstp, ones_val)
    c0, c1 = cnt[:NP], cnt[NP:]
    cnt_specs = [_row_spec(128), _row_spec(128)]

    g0 = _call_rows(_tc_scale, cnt_specs + [_row_spec(in_c)], in_c, (c0, c1, xp))

    a = scat_big(g0, srcp, dstp)
    g1 = _call_rows(
        _tc_mid,
        cnt_specs + [_row_spec(in_c)] * 3,
        in_c,
        (c0, c1, a[:NP], a[NP:], g0),
    )

    a = scat_big(g1, srcp, dstp)
    g2 = _call_rows(
        _tc_mlp,
        cnt_specs
        + [_row_spec(in_c)] * 3
        + [_full_spec(in_c, hid_c), _full_spec(1, hid_c), _full_spec(hid_c, hid_c)],
        hid_c,
        (c0, c1, a[:NP], a[NP:], g1, W1, b1.reshape(1, hid_c), W2p),
    )

    a = scat_big(g2, srcp, dstp)
    g3 = _call_rows(
        _tc_mid,
        cnt_specs + [_row_spec(hid_c)] * 3,
        hid_c,
        (c0, c1, a[:NP], a[NP:], g2),
    )

    a = scat_big(g3, srcp, dstp)
    out = _call_rows(
        _tc_final,
        cnt_specs + [_row_spec(hid_c)] * 3 + [_full_spec(1, out_c)],
        out_c,
        (c0, c1, a[:NP], a[NP:], g3, b2.reshape(1, out_c)),
    )
    return out[:N]


# speedup vs baseline: 1.3258x; 1.3258x over previous
import functools

import jax
import jax.numpy as jnp
from jax import lax
from jax.experimental import pallas as pl
from jax.experimental.pallas import tpu as pltpu
from jax.experimental.pallas import tpu_sc as plsc

NP = 10240
CH = 128
NCORES = 2
NSUB = 16
NW = NCORES * NSUB
ROWS_PER_TILE = NP // NSUB
ZR = 64


def _sc_mesh():
    return plsc.VectorSubcoreMesh(core_axis_name="c", subcore_axis_name="s")


def _zero_acc(acc, zbuf, sid, c):
    zvec = jnp.zeros((16,), jnp.float32)

    def fill(r, _):
        for t in range(c // 16):
            zbuf[r, pl.ds(t * 16, 16)] = zvec
        return 0

    lax.fori_loop(0, ZR, fill, 0)
    base_row = sid * ROWS_PER_TILE
    for t in range(ROWS_PER_TILE // ZR):
        pltpu.sync_copy(zbuf, acc.at[pl.ds(base_row + t * ZR, ZR)])


def _make_deg_kernel(EP):
    NCH = EP // (NW * CH)

    @functools.partial(
        pl.kernel,
        out_type=jax.ShapeDtypeStruct((NCORES * NP, 128), jnp.float32),
        mesh=_sc_mesh(),
        scratch_types=[
            pltpu.VMEM_SHARED((NP, 128), jnp.float32),
            pltpu.VMEM((CH,), jnp.int32),
            pltpu.VMEM((CH, 128), jnp.float32),
            pltpu.VMEM((ZR, 128), jnp.float32),
        ],
    )
    def k(dst_hbm, val_hbm, out_hbm, acc, dstb, valb, zbuf):
        cid = lax.axis_index("c")
        sid = lax.axis_index("s")
        _zero_acc(acc, zbuf, sid, 128)
        pltpu.sync_copy(val_hbm, valb)
        wid = sid * NCORES + cid
        ebase = wid * NCH * CH
        plsc.subcore_barrier()

        def body(ci, _):
            off = ebase + ci * CH
            pltpu.sync_copy(dst_hbm.at[pl.ds(off, CH)], dstb)
            pltpu.sync_copy(valb, acc.at[dstb], add=True)
            return 0

        lax.fori_loop(0, NCH, body, 0)
        plsc.subcore_barrier()
        base_row = sid * ROWS_PER_TILE
        pltpu.sync_copy(
            acc.at[pl.ds(base_row, ROWS_PER_TILE)],
            out_hbm.at[pl.ds(cid * NP + base_row, ROWS_PER_TILE)],
        )

    return k


def _make_scatter_kernel(C, EP):
    NCH = EP // (NW * CH)

    @functools.partial(
        pl.kernel,
        out_type=jax.ShapeDtypeStruct((NCORES * NP, C), jnp.float32),
        mesh=_sc_mesh(),
        scratch_types=[
            pltpu.VMEM_SHARED((NP, C), jnp.float32),
            pltpu.VMEM((CH,), jnp.int32),
            pltpu.VMEM((CH,), jnp.int32),
            pltpu.VMEM((CH, C), jnp.float32),
            pltpu.VMEM((ZR, C), jnp.float32),
            pltpu.SemaphoreType.DMA,
        ],
    )
    def k(g_hbm, src_hbm, dst_hbm, out_hbm, acc, srcb, dstb, rows, zbuf, sem):
        cid = lax.axis_index("c")
        sid = lax.axis_index("s")
        _zero_acc(acc, zbuf, sid, C)
        wid = sid * NCORES + cid
        ebase = wid * NCH * CH
        plsc.subcore_barrier()

        def body(ci, _):
            off = ebase + ci * CH
            pltpu.sync_copy(src_hbm.at[pl.ds(off, CH)], srcb)
            pltpu.sync_copy(dst_hbm.at[pl.ds(off, CH)], dstb)
            pltpu.async_copy(g_hbm.at[srcb], rows, sem).wait()
            pltpu.sync_copy(rows, acc.at[dstb], add=True)
            return 0

        lax.fori_loop(0, NCH, body, 0)
        plsc.subcore_barrier()
        base_row = sid * ROWS_PER_TILE
        pltpu.sync_copy(
            acc.at[pl.ds(base_row, ROWS_PER_TILE)],
            out_hbm.at[pl.ds(cid * NP + base_row, ROWS_PER_TILE)],
        )

    return k


_R = 256


def _dinv(c0, c1):
    return lax.rsqrt(c0[:, 0:1] + c1[:, 0:1] + 1.0)


def _tc_scale(c0_ref, c1_ref, x_ref, o_ref):
    o_ref[...] = x_ref[...] * _dinv(c0_ref[...], c1_ref[...])


def _tc_mid(c0_ref, c1_ref, a0_ref, a1_ref, g_ref, o_ref):
    d = _dinv(c0_ref[...], c1_ref[...])
    o_ref[...] = (d * d) * (a0_ref[...] + a1_ref[...] + g_ref[...])


def _tc_mlp(c0_ref, c1_ref, a0_ref, a1_ref, g_ref, w1_ref, b1_ref, w2_ref, o_ref):
    d = _dinv(c0_ref[...], c1_ref[...])
    p = d * (a0_ref[...] + a1_ref[...] + g_ref[...])
    hid = jnp.maximum(
        jnp.dot(p, w1_ref[...], preferred_element_type=jnp.float32) + b1_ref[...], 0.0
    )
    o_ref[...] = d * jnp.dot(hid, w2_ref[...], preferred_element_type=jnp.float32)


def _tc_final(c0_ref, c1_ref, a0_ref, a1_ref, g_ref, b2_ref, o_ref):
    d = _dinv(c0_ref[...], c1_ref[...])
    out_c = b2_ref.shape[1]
    s = (d * (a0_ref[...] + a1_ref[...] + g_ref[...]))[:, :out_c] + b2_ref[...]
    m = jnp.max(s, axis=1, keepdims=True)
    e = jnp.exp(s - m)
    lse = jnp.log(jnp.sum(e, axis=1, keepdims=True))
    o_ref[...] = s - m - lse


def _row_spec(c):
    return pl.BlockSpec((_R, c), lambda i: (i, 0))


def _full_spec(r, c):
    return pl.BlockSpec((r, c), lambda i: (0, 0))


def _call_rows(body, in_specs, out_c, args):
    grid = NP // _R
    return pl.pallas_call(
        body,
        grid=(grid,),
        in_specs=in_specs,
        out_specs=_row_spec(out_c),
        out_shape=jax.ShapeDtypeStruct((NP, out_c), jnp.float32),
    )(*args)


def kernel(x, edge_index, W1, b1, W2, b2):
    N, in_c = x.shape
    E = edge_index.shape[1]
    hid_c = W1.shape[1]
    out_c = W2.shape[1]

    EG = NW * CH
    EP = ((E + EG - 1) // EG) * EG

    xp = jnp.zeros((NP, in_c), jnp.float32).at[:N].set(x)
    pad = jnp.full((EP - E,), N, jnp.int32)
    srcp = jnp.concatenate([edge_index[0], pad])
    dstp = jnp.concatenate([edge_index[1], pad])
    W2p = jnp.zeros((hid_c, hid_c), jnp.float32).at[:, :out_c].set(W2)

    deg_k = _make_deg_kernel(EP)
    scat_big = _make_scatter_kernel(in_c, EP)

    ones_val = jnp.zeros((CH, 128), jnp.float32).at[:, 0].set(1.0)
    cnt = deg_k(dstp, ones_val)
    c0, c1 = cnt[:NP], cnt[NP:]
    cnt_specs = [_row_spec(128), _row_spec(128)]

    g0 = _call_rows(_tc_scale, cnt_specs + [_row_spec(in_c)], in_c, (c0, c1, xp))

    a = scat_big(g0, srcp, dstp)
    g1 = _call_rows(
        _tc_mid,
        cnt_specs + [_row_spec(in_c)] * 3,
        in_c,
        (c0, c1, a[:NP], a[NP:], g0),
    )

    a = scat_big(g1, srcp, dstp)
    g2 = _call_rows(
        _tc_mlp,
        cnt_specs
        + [_row_spec(in_c)] * 3
        + [_full_spec(in_c, hid_c), _full_spec(1, hid_c), _full_spec(hid_c, hid_c)],
        hid_c,
        (c0, c1, a[:NP], a[NP:], g1, W1, b1.reshape(1, hid_c), W2p),
    )

    a = scat_big(g2, srcp, dstp)
    g3 = _call_rows(
        _tc_mid,
        cnt_specs + [_row_spec(hid_c)] * 3,
        hid_c,
        (c0, c1, a[:NP], a[NP:], g2),
    )

    a = scat_big(g3, srcp, dstp)
    out = _call_rows(
        _tc_final,
        cnt_specs + [_row_spec(hid_c)] * 3 + [_full_spec(1, out_c)],
        out_c,
        (c0, c1, a[:NP], a[NP:], g3, b2.reshape(1, out_c)),
    )
    return out[:N]
